# baseline (speedup 1.0000x reference)
import functools

import jax
import jax.numpy as jnp
from jax import lax
from jax.experimental import pallas as pl
from jax.experimental.pallas import tpu as pltpu
from jax.experimental.pallas import tpu_sc as plsc

_SCALE = 10.0
_D = 32
_LANES = 16
_NW = 32
_BLK = 512


def _transpose_scale(wT, n_rows):
    n_full = n_rows // _BLK
    tail = n_rows - n_full * _BLK
    t_per_w = n_full // _NW
    rem = n_full - t_per_w * _NW
    mesh = plsc.VectorSubcoreMesh(core_axis_name="c", subcore_axis_name="s")
    n_cores = mesh.num_cores

    @functools.partial(
        pl.kernel,
        out_type=jax.ShapeDtypeStruct((n_rows * _D,), jnp.float32),
        mesh=mesh,
        scratch_types=[
            pltpu.VMEM((_D, _BLK), jnp.float32),
            pltpu.VMEM((_D, _BLK), jnp.float32),
            pltpu.VMEM((_BLK * _D,), jnp.float32),
            pltpu.VMEM((_BLK * _D,), jnp.float32),
            pltpu.VMEM((_D, tail), jnp.float32),
            pltpu.VMEM((tail * _D,), jnp.float32),
            pltpu.SemaphoreType.DMA,
            pltpu.SemaphoreType.DMA,
            pltpu.SemaphoreType.DMA,
            pltpu.SemaphoreType.DMA,
        ],
        compiler_params=pltpu.CompilerParams(
            use_tc_tiling_on_sc=True, needs_layout_passes=False
        ),
    )
    def body(w_hbm, out_hbm, bufa, bufb, obufa, obufb, tbuf, tobuf,
             isem0, isem1, osem0, osem1):
        wid = lax.axis_index("s") * n_cores + lax.axis_index("c")
        iota = lax.iota(jnp.int32, _LANES)
        lo16 = iota
        hi16 = iota + _LANES
        bufs = (bufa, bufb)
        obufs = (obufa, obufb)
        isems = (isem0, isem1)
        osems = (osem0, osem1)

        def c0_of(t):
            return (wid + t * _NW) * _BLK

        def transpose_block(src, dst, width):
            @plsc.parallel_loop(0, width, unroll=8)
            def _(c):
                zc = iota * 0 + c
                va = plsc.load_gather(src, [lo16, zc])
                vb = plsc.load_gather(src, [hi16, zc])
                dst[pl.ds(c * _D, _LANES)] = va * _SCALE
                dst[pl.ds(c * _D + _LANES, _LANES)] = vb * _SCALE

        din = [None] * t_per_w
        dout = [None] * t_per_w

        def start_in(t):
            p = t % 2
            return pltpu.async_copy(
                w_hbm.at[:, pl.ds(c0_of(t), _BLK)], bufs[p], isems[p]
            )

        din[0] = start_in(0)
        if t_per_w > 1:
            din[1] = start_in(1)
        for t in range(t_per_w):
            p = t % 2
            din[t].wait()
            if t >= 2:
                dout[t - 2].wait()
            transpose_block(bufs[p], obufs[p], _BLK)
            dout[t] = pltpu.async_copy(
                obufs[p], out_hbm.at[pl.ds(c0_of(t) * _D, _BLK * _D)],
                osems[p],
            )
            if t + 2 < t_per_w:
                din[t + 2] = start_in(t + 2)
        for t in range(max(t_per_w - 2, 0), t_per_w):
            dout[t].wait()

        def do_block_sync(c0, src, dst, width):
            pltpu.sync_copy(w_hbm.at[:, pl.ds(c0, width)], src)
            transpose_block(src, dst, width)
            pltpu.sync_copy(dst, out_hbm.at[pl.ds(c0 * _D, width * _D)])

        for r in range(rem):

            @pl.when(wid == r)
            def _():
                do_block_sync((t_per_w * _NW + r) * _BLK, bufa, obufa, _BLK)

        if tail:

            @pl.when(wid == rem)
            def _():
                do_block_sync(n_full * _BLK, tbuf, tobuf, tail)

    return body(wT)


def _gather_rows(x_flat, table_flat, *, b_per_w, chunk):
    n_chunks = b_per_w // chunk
    num_b = x_flat.shape[0]
    mesh = plsc.VectorSubcoreMesh(core_axis_name="c", subcore_axis_name="s")
    n_cores = mesh.num_cores
    table = table_flat.reshape(table_flat.shape[0] // _D, _D)

    @functools.partial(
        pl.kernel,
        out_type=jax.ShapeDtypeStruct((num_b, _D), jnp.float32),
        mesh=mesh,
        scratch_types=[
            pltpu.VMEM((b_per_w,), jnp.int32),
            pltpu.VMEM((chunk, _D), jnp.float32),
            pltpu.VMEM((chunk, _D), jnp.float32),
            pltpu.SemaphoreType.DMA,
            pltpu.SemaphoreType.DMA,
        ],
        compiler_params=pltpu.CompilerParams(use_tc_tiling_on_sc=False),
    )
    def body(x_hbm, w_hbm, out_hbm, idx_v, buf0, buf1, sem0, sem1):
        wid = lax.axis_index("s") * n_cores + lax.axis_index("c")
        base = wid * b_per_w
        pltpu.sync_copy(x_hbm.at[pl.ds(base, b_per_w)], idx_v)

        bufs = (buf0, buf1)
        sems = (sem0, sem1)

        def start_gather(g):
            p = g % 2
            return pltpu.async_copy(
                w_hbm.at[idx_v.at[pl.ds(g * chunk, chunk)]], bufs[p], sems[p]
            )

        descs = [None] * n_chunks
        descs[0] = start_gather(0)
        if n_chunks > 1:
            descs[1] = start_gather(1)

        for g in range(n_chunks):
            p = g % 2
            descs[g].wait()
            pltpu.sync_copy(
                bufs[p], out_hbm.at[pl.ds(base + g * chunk, chunk)]
            )
            if g + 2 < n_chunks:
                descs[g + 2] = start_gather(g + 2)

    return body(x_flat, table)


@jax.jit
def _scaled_embedding(x, weight):
    b0, b1 = x.shape
    num_b = b0 * b1
    x_flat = x.reshape(num_b)
    if x_flat.dtype != jnp.int32:
        x_flat = x_flat.astype(jnp.int32)
    table_flat = _transpose_scale(weight.T, weight.shape[0])
    out = _gather_rows(
        x_flat, table_flat, b_per_w=num_b // _NW, chunk=1664
    )
    return out.reshape(b0, b1, _D)


def kernel(x, weight):
    return _scaled_embedding(x, weight)


# speedup vs baseline: 1.0111x; 1.0111x over previous
import functools

import jax
import jax.numpy as jnp
from jax import lax
from jax.experimental import pallas as pl
from jax.experimental.pallas import tpu as pltpu
from jax.experimental.pallas import tpu_sc as plsc

_SCALE = 10.0
_D = 32
_LANES = 16
_NW = 32
_BLK = 512


def _transpose_scale(wT, n_rows):
    n_full = n_rows // _BLK
    tail = n_rows - n_full * _BLK
    t_per_w = n_full // _NW
    rem = n_full - t_per_w * _NW
    mesh = plsc.VectorSubcoreMesh(core_axis_name="c", subcore_axis_name="s")
    n_cores = mesh.num_cores

    @functools.partial(
        pl.kernel,
        out_type=jax.ShapeDtypeStruct((n_rows * _D,), jnp.float32),
        mesh=mesh,
        scratch_types=[
            pltpu.VMEM((_D, _BLK + 1), jnp.float32),
            pltpu.VMEM((_D, _BLK + 1), jnp.float32),
            pltpu.VMEM((_BLK * _D,), jnp.float32),
            pltpu.VMEM((_BLK * _D,), jnp.float32),
            pltpu.VMEM((_D, tail), jnp.float32),
            pltpu.VMEM((tail * _D,), jnp.float32),
            pltpu.SemaphoreType.DMA,
            pltpu.SemaphoreType.DMA,
            pltpu.SemaphoreType.DMA,
            pltpu.SemaphoreType.DMA,
        ],
        compiler_params=pltpu.CompilerParams(
            use_tc_tiling_on_sc=True, needs_layout_passes=False
        ),
    )
    def body(w_hbm, out_hbm, bufa, bufb, obufa, obufb, tbuf, tobuf,
             isem0, isem1, osem0, osem1):
        wid = lax.axis_index("s") * n_cores + lax.axis_index("c")
        iota = lax.iota(jnp.int32, _LANES)
        lo16 = iota
        hi16 = iota + _LANES
        bufs = (bufa, bufb)
        obufs = (obufa, obufb)
        isems = (isem0, isem1)
        osems = (osem0, osem1)

        def c0_of(t):
            return (wid + t * _NW) * _BLK

        def transpose_block(src, dst, width):
            @plsc.parallel_loop(0, width, unroll=4)
            def _(c):
                zc = iota * 0 + c
                va = plsc.load_gather(src, [lo16, zc])
                vb = plsc.load_gather(src, [hi16, zc])
                dst[pl.ds(c * _D, _LANES)] = va * _SCALE
                dst[pl.ds(c * _D + _LANES, _LANES)] = vb * _SCALE

        din = [None] * t_per_w
        dout = [None] * t_per_w

        def start_in(t):
            p = t % 2
            return pltpu.async_copy(
                w_hbm.at[:, pl.ds(c0_of(t), _BLK)],
                bufs[p].at[:, pl.ds(0, _BLK)], isems[p]
            )

        din[0] = start_in(0)
        if t_per_w > 1:
            din[1] = start_in(1)
        for t in range(t_per_w):
            p = t % 2
            din[t].wait()
            if t >= 2:
                dout[t - 2].wait()
            transpose_block(bufs[p], obufs[p], _BLK)
            dout[t] = pltpu.async_copy(
                obufs[p], out_hbm.at[pl.ds(c0_of(t) * _D, _BLK * _D)],
                osems[p],
            )
            if t + 2 < t_per_w:
                din[t + 2] = start_in(t + 2)
        for t in range(max(t_per_w - 2, 0), t_per_w):
            dout[t].wait()

        def do_block_sync(c0, src, dst, width, w_src):
            pltpu.sync_copy(
                w_hbm.at[:, pl.ds(c0, width)], src.at[:, pl.ds(0, w_src)]
            )
            transpose_block(src, dst, width)
            pltpu.sync_copy(dst, out_hbm.at[pl.ds(c0 * _D, width * _D)])

        for r in range(rem):

            @pl.when(wid == r)
            def _():
                do_block_sync((t_per_w * _NW + r) * _BLK, bufa, obufa, _BLK, _BLK)

        if tail:

            @pl.when(wid == rem)
            def _():
                do_block_sync(n_full * _BLK, tbuf, tobuf, tail, tail)

    return body(wT)


def _gather_rows(x_flat, table_flat, *, b_per_w, chunk):
    n_chunks = b_per_w // chunk
    num_b = x_flat.shape[0]
    mesh = plsc.VectorSubcoreMesh(core_axis_name="c", subcore_axis_name="s")
    n_cores = mesh.num_cores
    table = table_flat.reshape(table_flat.shape[0] // _D, _D)

    @functools.partial(
        pl.kernel,
        out_type=jax.ShapeDtypeStruct((num_b, _D), jnp.float32),
        mesh=mesh,
        scratch_types=[
            pltpu.VMEM((b_per_w,), jnp.int32),
            pltpu.VMEM((chunk, _D), jnp.float32),
            pltpu.VMEM((chunk, _D), jnp.float32),
            pltpu.SemaphoreType.DMA,
            pltpu.SemaphoreType.DMA,
        ],
        compiler_params=pltpu.CompilerParams(use_tc_tiling_on_sc=False),
    )
    def body(x_hbm, w_hbm, out_hbm, idx_v, buf0, buf1, sem0, sem1):
        wid = lax.axis_index("s") * n_cores + lax.axis_index("c")
        base = wid * b_per_w
        pltpu.sync_copy(x_hbm.at[pl.ds(base, b_per_w)], idx_v)

        bufs = (buf0, buf1)
        sems = (sem0, sem1)

        def start_gather(g):
            p = g % 2
            return pltpu.async_copy(
                w_hbm.at[idx_v.at[pl.ds(g * chunk, chunk)]], bufs[p], sems[p]
            )

        descs = [None] * n_chunks
        descs[0] = start_gather(0)
        if n_chunks > 1:
            descs[1] = start_gather(1)

        for g in range(n_chunks):
            p = g % 2
            descs[g].wait()
            pltpu.sync_copy(
                bufs[p], out_hbm.at[pl.ds(base + g * chunk, chunk)]
            )
            if g + 2 < n_chunks:
                descs[g + 2] = start_gather(g + 2)

    return body(x_flat, table)


@jax.jit
def _scaled_embedding(x, weight):
    b0, b1 = x.shape
    num_b = b0 * b1
    x_flat = x.reshape(num_b)
    if x_flat.dtype != jnp.int32:
        x_flat = x_flat.astype(jnp.int32)
    table_flat = _transpose_scale(weight.T, weight.shape[0])
    out = _gather_rows(
        x_flat, table_flat, b_per_w=num_b // _NW, chunk=1664
    )
    return out.reshape(b0, b1, _D)


def kernel(x, weight):
    return _scaled_embedding(x, weight)


# speedup vs baseline: 1.0499x; 1.0384x over previous
import functools

import jax
import jax.numpy as jnp
from jax import lax
from jax.experimental import pallas as pl
from jax.experimental.pallas import tpu as pltpu
from jax.experimental.pallas import tpu_sc as plsc

_SCALE = 10.0
_D = 32
_LANES = 16


@functools.partial(jax.jit, static_argnames=("b_per_w", "chunk", "n_workers"))
def _scaled_embedding(x_flat, weight, *, b_per_w, chunk, n_workers):
    n_chunks = b_per_w // chunk
    num_b = x_flat.shape[0]
    mesh = plsc.VectorSubcoreMesh(core_axis_name="c", subcore_axis_name="s")
    n_cores = mesh.num_cores

    @functools.partial(
        pl.kernel,
        out_type=jax.ShapeDtypeStruct((num_b, _D), jnp.float32),
        mesh=mesh,
        scratch_types=[
            pltpu.VMEM((b_per_w,), jnp.int32),
            pltpu.VMEM((chunk, _D), jnp.float32),
            pltpu.VMEM((chunk, _D), jnp.float32),
            pltpu.SemaphoreType.DMA,
            pltpu.SemaphoreType.DMA,
        ],
        compiler_params=pltpu.CompilerParams(use_tc_tiling_on_sc=False),
    )
    def body(x_hbm, w_hbm, out_hbm, idx_v, buf0, buf1, sem0, sem1):
        wid = lax.axis_index("s") * n_cores + lax.axis_index("c")
        base = wid * b_per_w
        pltpu.sync_copy(x_hbm.at[pl.ds(base, b_per_w)], idx_v)

        bufs = (buf0, buf1)
        sems = (sem0, sem1)

        def start_gather(g):
            p = g % 2
            return pltpu.async_copy(
                w_hbm.at[idx_v.at[pl.ds(g * chunk, chunk)]], bufs[p], sems[p]
            )

        descs = [None] * n_chunks
        descs[0] = start_gather(0)
        if n_chunks > 1:
            descs[1] = start_gather(1)

        for g in range(n_chunks):
            p = g % 2
            buf = bufs[p]
            descs[g].wait()

            @plsc.parallel_loop(0, chunk, unroll=8)
            def _(i, _buf=buf):
                _buf[i, pl.ds(0, _LANES)] = _buf[i, pl.ds(0, _LANES)] * _SCALE
                _buf[i, pl.ds(_LANES, _LANES)] = (
                    _buf[i, pl.ds(_LANES, _LANES)] * _SCALE
                )

            pltpu.sync_copy(buf, out_hbm.at[pl.ds(base + g * chunk, chunk)])
            if g + 2 < n_chunks:
                descs[g + 2] = start_gather(g + 2)

    return body(x_flat, weight)


def kernel(x, weight):
    b0, b1 = x.shape
    num_b = b0 * b1
    x_flat = x.reshape(num_b)
    if x_flat.dtype != jnp.int32:
        x_flat = x_flat.astype(jnp.int32)
    n_workers = 32
    b_per_w = num_b // n_workers
    out = _scaled_embedding(
        x_flat, weight, b_per_w=b_per_w, chunk=1664, n_workers=n_workers
    )
    return out.reshape(b0, b1, _D)
